# baseline (speedup 1.0000x reference)
import functools

import jax
import jax.numpy as jnp
from jax import lax
from jax.experimental import pallas as pl
from jax.experimental.pallas import tpu as pltpu
from jax.experimental.pallas import tpu_sc as plsc

NC = 2
NS = 16
NW = NC * NS
CHUNK = 128


def kernel(storage, vals, next_ptr):
    cap, dim = storage.shape
    batch = vals.shape[0]
    next_ptr = jnp.asarray(next_ptr, jnp.int32)
    positions = (next_ptr + jnp.arange(batch, dtype=jnp.int32)) % cap

    rows_per_w = batch // NW
    n_chunks = rows_per_w // CHUNK
    pos3 = positions.reshape(NW, n_chunks, CHUNK)

    mesh = plsc.VectorSubcoreMesh(core_axis_name="c", subcore_axis_name="s")

    @functools.partial(
        pl.kernel,
        mesh=mesh,
        scratch_types=[
            pltpu.VMEM((n_chunks, CHUNK), jnp.int32),
            pltpu.VMEM((rows_per_w, dim), jnp.float32),
            pltpu.SemaphoreType.DMA,
        ],
        compiler_params=pltpu.CompilerParams(use_tc_tiling_on_sc=False),
    )
    def sc_scatter(out_ref, vals_hbm, pos_hbm, idx_v, rows_v, sem):
        wid = lax.axis_index("s") * NC + lax.axis_index("c")
        base = wid * rows_per_w
        pltpu.sync_copy(vals_hbm.at[pl.ds(base, rows_per_w)], rows_v)
        pltpu.sync_copy(pos_hbm.at[wid], idx_v)
        copies = []
        for j in range(n_chunks):
            copies.append(
                pltpu.async_copy(
                    rows_v.at[pl.ds(j * CHUNK, CHUNK)],
                    out_ref.at[idx_v.at[j]],
                    sem,
                )
            )
        for c in copies:
            c.wait()

    out_ref = jax.new_ref(storage)
    sc_scatter(out_ref, vals, pos3)
    new_storage = out_ref[...]
    new_ptr = (next_ptr + batch) % cap
    return new_storage, new_ptr.astype(jnp.int32)
